# Initial kernel scaffold; baseline (speedup 1.0000x reference)
import jax
import jax.numpy as jnp
from jax.experimental import pallas as pl


def kernel(inputs, keys, vals):
    raise NotImplementedError("write your pallas kernel here")


# speedup vs baseline: 1826.9212x; 1826.9212x over previous
import functools

import jax
import jax.numpy as jnp
from jax import lax
from jax.experimental import pallas as pl
from jax.experimental.pallas import tpu as pltpu
from jax.experimental.pallas import tpu_sc as plsc

VOCAB = 1000
LUT_SIZE = 1024
MASK_VALUE = 0
DEFAULT_VAL = 1

NC, NS, L = 2, 16, 16
NW = NC * NS

ROWS, COLS = 16384, 200
TOTAL = ROWS * COLS
PER_W = TOTAL // NW


def _body(inputs_hbm, keys_hbm, vals_hbm, out_hbm, keys_v, vals_v, lut_v, chunk_v):
    wid = lax.axis_index("s") * NC + lax.axis_index("c")
    base = wid * PER_W

    pltpu.sync_copy(keys_hbm, keys_v.at[pl.ds(0, VOCAB)])
    pltpu.sync_copy(vals_hbm, vals_v.at[pl.ds(0, VOCAB)])

    lane = lax.iota(jnp.int32, L)
    default = jnp.full((L,), DEFAULT_VAL, jnp.int32)

    def init_body(i, c):
        lut_v[pl.ds(i * L, L)] = default
        return c

    lax.fori_loop(0, LUT_SIZE // L, init_body, 0)

    def scat_body(j, c):
        k = keys_v[pl.ds(j * L, L)]
        v = vals_v[pl.ds(j * L, L)]
        valid = (j * L + lane < VOCAB) & (k >= 0) & (k < LUT_SIZE)
        plsc.store_scatter(lut_v, [k], v, mask=valid)
        return c

    lax.fori_loop(0, LUT_SIZE // L, scat_body, 0)

    head = lut_v[pl.ds(0, L)]
    lut_v[pl.ds(0, L)] = jnp.where(lane == MASK_VALUE, 0, head)

    pltpu.sync_copy(inputs_hbm.at[pl.ds(base, PER_W)], chunk_v)

    def gather_body(i, c):
        x = chunk_v[pl.ds(i * L, L)]
        chunk_v[pl.ds(i * L, L)] = plsc.load_gather(lut_v, [x])
        return c

    lax.fori_loop(0, PER_W // L, gather_body, 0)

    pltpu.sync_copy(chunk_v, out_hbm.at[pl.ds(base, PER_W)])


_lookup = functools.partial(
    pl.kernel,
    out_type=jax.ShapeDtypeStruct((TOTAL,), jnp.int32),
    mesh=plsc.VectorSubcoreMesh(core_axis_name="c", subcore_axis_name="s"),
    scratch_types=[
        pltpu.VMEM((LUT_SIZE,), jnp.int32),
        pltpu.VMEM((LUT_SIZE,), jnp.int32),
        pltpu.VMEM((LUT_SIZE,), jnp.int32),
        pltpu.VMEM((PER_W,), jnp.int32),
    ],
    compiler_params=pltpu.CompilerParams(needs_layout_passes=False),
)(_body)


@jax.jit
def kernel(inputs, keys, vals):
    out = _lookup(inputs.reshape(TOTAL), keys, vals)
    return out.reshape(inputs.shape)


# speedup vs baseline: 2786.3345x; 1.5252x over previous
import functools

import jax
import jax.numpy as jnp
from jax import lax
from jax.experimental import pallas as pl
from jax.experimental.pallas import tpu as pltpu
from jax.experimental.pallas import tpu_sc as plsc

VOCAB = 1000
LUT_SIZE = 1024
MASK_VALUE = 0
DEFAULT_VAL = 1

NC, NS, L = 2, 16, 16
NW = NC * NS

ROWS, COLS = 16384, 200
TOTAL = ROWS * COLS
PER_W = TOTAL // NW
NCHUNK = 8
C = PER_W // NCHUNK


def _body(inputs_hbm, keys_hbm, vals_hbm, out_hbm,
          keys_v, vals_v, lut_v, chunk_v, sem_in0, sem_in1, sem_out):
    wid = lax.axis_index("s") * NC + lax.axis_index("c")
    base = wid * PER_W
    sems_in = (sem_in0, sem_in1)

    in_descs = [pltpu.async_copy(inputs_hbm.at[pl.ds(base, C)],
                                 chunk_v.at[pl.ds(0, C)], sems_in[0])]

    pltpu.sync_copy(keys_hbm, keys_v.at[pl.ds(0, VOCAB)])
    pltpu.sync_copy(vals_hbm, vals_v.at[pl.ds(0, VOCAB)])

    lane = lax.iota(jnp.int32, L)
    default = jnp.full((L,), DEFAULT_VAL, jnp.int32)

    @plsc.parallel_loop(0, LUT_SIZE // L)
    def _init(i):
        lut_v[pl.ds(i * L, L)] = default

    def _scat(j, c):
        k = keys_v[pl.ds(j * L, L)]
        v = vals_v[pl.ds(j * L, L)]
        valid = (j * L + lane < VOCAB) & (k >= 0) & (k < LUT_SIZE)
        plsc.store_scatter(lut_v, [k], v, mask=valid)
        return c

    lax.fori_loop(0, LUT_SIZE // L, _scat, 0)

    head = lut_v[pl.ds(0, L)]
    lut_v[pl.ds(0, L)] = jnp.where(lane == MASK_VALUE, 0, head)

    out_descs = []
    for g in range(NCHUNK):
        if g + 1 < NCHUNK:
            in_descs.append(pltpu.async_copy(
                inputs_hbm.at[pl.ds(base + (g + 1) * C, C)],
                chunk_v.at[pl.ds((g + 1) * C, C)], sems_in[(g + 1) % 2]))
        in_descs[g].wait()
        off = g * C

        @plsc.parallel_loop(0, C // L, unroll=8)
        def _gather(i, off=off):
            x = chunk_v[pl.ds(off + i * L, L)]
            chunk_v[pl.ds(off + i * L, L)] = plsc.load_gather(lut_v, [x])

        out_descs.append(pltpu.async_copy(
            chunk_v.at[pl.ds(off, C)],
            out_hbm.at[pl.ds(base + off, C)], sem_out))

    for d in out_descs:
        d.wait()


_lookup = functools.partial(
    pl.kernel,
    out_type=jax.ShapeDtypeStruct((TOTAL,), jnp.int32),
    mesh=plsc.VectorSubcoreMesh(core_axis_name="c", subcore_axis_name="s"),
    scratch_types=[
        pltpu.VMEM((LUT_SIZE,), jnp.int32),
        pltpu.VMEM((LUT_SIZE,), jnp.int32),
        pltpu.VMEM((LUT_SIZE,), jnp.int32),
        pltpu.VMEM((PER_W,), jnp.int32),
        pltpu.SemaphoreType.DMA,
        pltpu.SemaphoreType.DMA,
        pltpu.SemaphoreType.DMA,
    ],
    compiler_params=pltpu.CompilerParams(needs_layout_passes=False),
)(_body)


@jax.jit
def kernel(inputs, keys, vals):
    out = _lookup(inputs.reshape(TOTAL), keys, vals)
    return out.reshape(inputs.shape)


# speedup vs baseline: 8738.7693x; 3.1363x over previous
import functools

import jax
import jax.numpy as jnp
from jax import lax
from jax.experimental import pallas as pl
from jax.experimental.pallas import tpu as pltpu
from jax.experimental.pallas import tpu_sc as plsc

VOCAB = 1000
LUT_SIZE = 1024
MASK_VALUE = 0
DEFAULT_VAL = 1

NC, NS, L = 2, 16, 16
NW = NC * NS

ROWS, COLS = 16384, 200
TR, TC = COLS, ROWS
MAX_RPW = -(-TR // NW)
NBIG = NW * MAX_RPW - TR
NSMALL = NW - NBIG


def _body(inputs_hbm, keys_hbm, vals_hbm, out_hbm,
          keys_v, vals_v, lut_v, bufs, sems_in, sem_out):
    wid = lax.axis_index("s") * NC + lax.axis_index("c")
    base_row = wid * MAX_RPW - jnp.maximum(wid - NSMALL, 0)
    nrows = jnp.where(wid < NSMALL, MAX_RPW, MAX_RPW - 1)

    pltpu.async_copy(inputs_hbm.at[base_row], bufs[0], sems_in[0])

    pltpu.sync_copy(keys_hbm, keys_v.at[pl.ds(0, VOCAB)])
    pltpu.sync_copy(vals_hbm, vals_v.at[pl.ds(0, VOCAB)])

    lane = lax.iota(jnp.int32, L)
    default = jnp.full((L,), DEFAULT_VAL, jnp.int32)

    @plsc.parallel_loop(0, LUT_SIZE // L)
    def _init(i):
        lut_v[pl.ds(i * L, L)] = default

    def _scat(j, c):
        k = keys_v[pl.ds(j * L, L)]
        v = vals_v[pl.ds(j * L, L)]
        valid = (j * L + lane < VOCAB) & (k >= 0) & (k < LUT_SIZE)
        plsc.store_scatter(lut_v, [k], v, mask=valid)
        return c

    lax.fori_loop(0, LUT_SIZE // L, _scat, 0)

    head = lut_v[pl.ds(0, L)]
    lut_v[pl.ds(0, L)] = jnp.where(lane == MASK_VALUE, 0, head)

    for t in range(MAX_RPW):
        if t + 1 < MAX_RPW:
            @pl.when(t + 1 < nrows)
            def _prefetch(t=t):
                pltpu.async_copy(inputs_hbm.at[base_row + t + 1],
                                 bufs[t + 1], sems_in[(t + 1) % 2])

        @pl.when(t < nrows)
        def _process(t=t):
            pltpu.make_async_copy(inputs_hbm.at[base_row + t],
                                  bufs[t], sems_in[t % 2]).wait()
            buf = bufs[t]

            @plsc.parallel_loop(0, TC // L, unroll=8)
            def _gather(i):
                x = buf[pl.ds(i * L, L)]
                buf[pl.ds(i * L, L)] = plsc.load_gather(lut_v, [x])

            pltpu.async_copy(buf, out_hbm.at[base_row + t], sem_out)

    for t in range(MAX_RPW):
        @pl.when(t < nrows)
        def _drain(t=t):
            pltpu.make_async_copy(bufs[t],
                                  out_hbm.at[base_row + t], sem_out).wait()


_lookup = functools.partial(
    pl.kernel,
    out_type=jax.ShapeDtypeStruct((TR, TC), jnp.int32),
    mesh=plsc.VectorSubcoreMesh(core_axis_name="c", subcore_axis_name="s"),
    scratch_types=[
        pltpu.VMEM((LUT_SIZE,), jnp.int32),
        pltpu.VMEM((LUT_SIZE,), jnp.int32),
        pltpu.VMEM((LUT_SIZE,), jnp.int32),
        [pltpu.VMEM((TC,), jnp.int32) for _ in range(MAX_RPW)],
        [pltpu.SemaphoreType.DMA for _ in range(2)],
        pltpu.SemaphoreType.DMA,
    ],
    compiler_params=pltpu.CompilerParams(needs_layout_passes=False),
)(_body)


@jax.jit
def kernel(inputs, keys, vals):
    out = _lookup(inputs.T, keys, vals)
    return out.T


# speedup vs baseline: 9212.1652x; 1.0542x over previous
import functools

import jax
import jax.numpy as jnp
from jax import lax
from jax.experimental import pallas as pl
from jax.experimental.pallas import tpu as pltpu
from jax.experimental.pallas import tpu_sc as plsc

VOCAB = 1000
LUT_SIZE = 1024
MASK_VALUE = 0
DEFAULT_VAL = 1

NC, NS, L = 2, 16, 16
NW = NC * NS

ROWS, COLS = 16384, 200
TR, TC = COLS, ROWS
FULL_RPW = TR // NW
REM_ROWS = TR - FULL_RPW * NW
QUART = TC * REM_ROWS // NW
NSLOT = FULL_RPW + 1
NBUF = 4


def _body(inputs_hbm, keys_hbm, vals_hbm, out_hbm,
          keys_v, vals_v, lut_v, bufs, sems_in, sem_out):
    wid = lax.axis_index("s") * NC + lax.axis_index("c")
    base_row = wid * FULL_RPW
    rem_row = FULL_RPW * NW + wid // (TC // QUART)
    rem_col = (wid % (TC // QUART)) * QUART

    def in_copy(s, make_only=False):
        mk = pltpu.make_async_copy if make_only else pltpu.async_copy
        if s < FULL_RPW:
            return mk(inputs_hbm.at[base_row + s], bufs[s % NBUF],
                      sems_in[s % 2])
        return mk(inputs_hbm.at[rem_row, pl.ds(rem_col, QUART)],
                  bufs[s % NBUF].at[pl.ds(0, QUART)], sems_in[s % 2])

    def out_copy(s, make_only=False):
        mk = pltpu.make_async_copy if make_only else pltpu.async_copy
        if s < FULL_RPW:
            return mk(bufs[s % NBUF], out_hbm.at[base_row + s], sem_out)
        return mk(bufs[s % NBUF].at[pl.ds(0, QUART)],
                  out_hbm.at[rem_row, pl.ds(rem_col, QUART)], sem_out)

    in_copy(0)

    pltpu.sync_copy(keys_hbm, keys_v.at[pl.ds(0, VOCAB)])
    pltpu.sync_copy(vals_hbm, vals_v.at[pl.ds(0, VOCAB)])

    lane = lax.iota(jnp.int32, L)
    default = jnp.full((L,), DEFAULT_VAL, jnp.int32)

    @plsc.parallel_loop(0, LUT_SIZE // L, unroll=4)
    def _init(i):
        lut_v[pl.ds(i * L, L)] = default

    @plsc.parallel_loop(0, LUT_SIZE // L, unroll=4)
    def _scat(j):
        k = keys_v[pl.ds(j * L, L)]
        v = vals_v[pl.ds(j * L, L)]
        valid = (j * L + lane < VOCAB) & (k >= 0) & (k < LUT_SIZE)
        plsc.store_scatter(lut_v, [k], v, mask=valid)

    head = lut_v[pl.ds(0, L)]
    lut_v[pl.ds(0, L)] = jnp.where(lane == MASK_VALUE, 0, head)

    for s in range(NSLOT):
        if s + 1 < NSLOT:
            if s + 1 >= NBUF:
                out_copy(s + 1 - NBUF, make_only=True).wait()
            in_copy(s + 1)

        in_copy(s, make_only=True).wait()
        buf = bufs[s % NBUF]
        n = TC if s < FULL_RPW else QUART

        @plsc.parallel_loop(0, n // L, unroll=8)
        def _gather(i, buf=buf):
            x = buf[pl.ds(i * L, L)]
            buf[pl.ds(i * L, L)] = plsc.load_gather(lut_v, [x])

        out_copy(s)

    for s in range(max(0, NSLOT - NBUF), NSLOT):
        out_copy(s, make_only=True).wait()


_lookup = functools.partial(
    pl.kernel,
    out_type=jax.ShapeDtypeStruct((TR, TC), jnp.int32),
    mesh=plsc.VectorSubcoreMesh(core_axis_name="c", subcore_axis_name="s"),
    scratch_types=[
        pltpu.VMEM((LUT_SIZE,), jnp.int32),
        pltpu.VMEM((LUT_SIZE,), jnp.int32),
        pltpu.VMEM((LUT_SIZE,), jnp.int32),
        [pltpu.VMEM((TC,), jnp.int32) for _ in range(NBUF)],
        [pltpu.SemaphoreType.DMA for _ in range(2)],
        pltpu.SemaphoreType.DMA,
    ],
    compiler_params=pltpu.CompilerParams(needs_layout_passes=False,
                                         skip_device_barrier=True),
)(_body)


@jax.jit
def kernel(inputs, keys, vals):
    out = _lookup(inputs.T, keys, vals)
    return out.T


# speedup vs baseline: 9242.2248x; 1.0033x over previous
import functools

import jax
import jax.numpy as jnp
from jax import lax
from jax.experimental import pallas as pl
from jax.experimental.pallas import tpu as pltpu
from jax.experimental.pallas import tpu_sc as plsc

VOCAB = 1000
LUT_SIZE = 1024
MASK_VALUE = 0
DEFAULT_VAL = 1

NC, NS, L = 2, 16, 16
NW = NC * NS

ROWS, COLS = 16384, 200
TR, TC = COLS, ROWS
FULL_RPW = TR // NW
REM_ROWS = TR - FULL_RPW * NW
QUART = TC * REM_ROWS // NW
NSLOT = FULL_RPW + 1
NBUF = 4


def _body(inputs_hbm, keys_hbm, vals_hbm, out_hbm,
          keys_v, vals_v, lut_v, bufs, sems_in, sem_out):
    wid = lax.axis_index("s") * NC + lax.axis_index("c")
    base_row = wid * FULL_RPW
    rem_row = FULL_RPW * NW + wid // (TC // QUART)
    rem_col = (wid % (TC // QUART)) * QUART

    def in_copy(s, make_only=False):
        mk = pltpu.make_async_copy if make_only else pltpu.async_copy
        if s < FULL_RPW:
            return mk(inputs_hbm.at[base_row + s], bufs[s % NBUF],
                      sems_in[s % 2])
        return mk(inputs_hbm.at[rem_row, pl.ds(rem_col, QUART)],
                  bufs[s % NBUF].at[pl.ds(0, QUART)], sems_in[s % 2])

    def out_copy(s, make_only=False):
        mk = pltpu.make_async_copy if make_only else pltpu.async_copy
        if s < FULL_RPW:
            return mk(bufs[s % NBUF], out_hbm.at[base_row + s], sem_out)
        return mk(bufs[s % NBUF].at[pl.ds(0, QUART)],
                  out_hbm.at[rem_row, pl.ds(rem_col, QUART)], sem_out)

    in_copy(0)

    pltpu.sync_copy(keys_hbm, keys_v.at[pl.ds(0, VOCAB)])
    pltpu.sync_copy(vals_hbm, vals_v.at[pl.ds(0, VOCAB)])

    lane = lax.iota(jnp.int32, L)
    default = jnp.full((L,), DEFAULT_VAL, jnp.int32)

    @plsc.parallel_loop(0, LUT_SIZE // L, unroll=4)
    def _init(i):
        lut_v[pl.ds(i * L, L)] = default

    @plsc.parallel_loop(0, LUT_SIZE // L, unroll=4)
    def _scat(j):
        k = keys_v[pl.ds(j * L, L)]
        v = vals_v[pl.ds(j * L, L)]
        valid = (j * L + lane < VOCAB) & (k >= 0) & (k < LUT_SIZE)
        plsc.store_scatter(lut_v, [k], v, mask=valid)

    head = lut_v[pl.ds(0, L)]
    lut_v[pl.ds(0, L)] = jnp.where(lane == MASK_VALUE, 0, head)

    for s in range(NSLOT):
        if s + 1 < NSLOT:
            if s + 1 >= NBUF:
                out_copy(s + 1 - NBUF, make_only=True).wait()
            in_copy(s + 1)

        in_copy(s, make_only=True).wait()
        buf = bufs[s % NBUF]
        n = TC if s < FULL_RPW else QUART

        @plsc.parallel_loop(0, n // L, unroll=8)
        def _gather(i, buf=buf):
            x = buf[pl.ds(i * L, L)]
            buf[pl.ds(i * L, L)] = plsc.load_gather(lut_v, [x])

        out_copy(s)

    for s in range(max(0, NSLOT - NBUF), NSLOT):
        out_copy(s, make_only=True).wait()


_lookup = functools.partial(
    pl.kernel,
    out_type=jax.ShapeDtypeStruct((TR, TC), jnp.int32),
    mesh=plsc.VectorSubcoreMesh(core_axis_name="c", subcore_axis_name="s"),
    scratch_types=[
        pltpu.VMEM((LUT_SIZE,), jnp.int32),
        pltpu.VMEM((LUT_SIZE,), jnp.int32),
        pltpu.VMEM((LUT_SIZE,), jnp.int32),
        [pltpu.VMEM((TC,), jnp.int32) for _ in range(NBUF)],
        [pltpu.SemaphoreType.DMA for _ in range(2)],
        pltpu.SemaphoreType.DMA,
    ],
    compiler_params=pltpu.CompilerParams(needs_layout_passes=False,
                                         skip_device_barrier=True,
                                         disable_bounds_checks=True,
                                         disable_semaphore_checks=True),
)(_body)


@jax.jit
def kernel(inputs, keys, vals):
    out = _lookup(inputs.T, keys, vals)
    return out.T


# speedup vs baseline: 9427.6535x; 1.0201x over previous
import functools

import jax
import jax.numpy as jnp
from jax import lax
from jax.experimental import pallas as pl
from jax.experimental.pallas import tpu as pltpu
from jax.experimental.pallas import tpu_sc as plsc

VOCAB = 1000
LUT_SIZE = 1024
MASK_VALUE = 0
DEFAULT_VAL = 1

NC, NS, L = 2, 16, 16
NW = NC * NS

ROWS, COLS = 16384, 200
TR, TC = COLS, ROWS
FULL_RPW = TR // NW
REM_ROWS = TR - FULL_RPW * NW
QUART = TC * REM_ROWS // NW
NSLOT = FULL_RPW + 1
NBUF = 4


def _body(inputs_hbm, keys_hbm, vals_hbm, out_hbm,
          keys_v, vals_v, lut_v, bufs, sems_in, sem_out):
    wid = lax.axis_index("s") * NC + lax.axis_index("c")
    base_row = wid * FULL_RPW
    rem_row = FULL_RPW * NW + wid // (TC // QUART)
    rem_col = (wid % (TC // QUART)) * QUART

    def in_copy(s, make_only=False):
        mk = pltpu.make_async_copy if make_only else pltpu.async_copy
        if s < FULL_RPW:
            return mk(inputs_hbm.at[base_row + s], bufs[s % NBUF],
                      sems_in[s % 3])
        return mk(inputs_hbm.at[rem_row, pl.ds(rem_col, QUART)],
                  bufs[s % NBUF].at[pl.ds(0, QUART)], sems_in[s % 3])

    def out_copy(s, make_only=False):
        mk = pltpu.make_async_copy if make_only else pltpu.async_copy
        if s < FULL_RPW:
            return mk(bufs[s % NBUF], out_hbm.at[base_row + s], sem_out)
        return mk(bufs[s % NBUF].at[pl.ds(0, QUART)],
                  out_hbm.at[rem_row, pl.ds(rem_col, QUART)], sem_out)

    in_copy(0)
    in_copy(1)

    pltpu.sync_copy(keys_hbm, keys_v.at[pl.ds(0, VOCAB)])
    pltpu.sync_copy(vals_hbm, vals_v.at[pl.ds(0, VOCAB)])

    lane = lax.iota(jnp.int32, L)
    default = jnp.full((L,), DEFAULT_VAL, jnp.int32)

    @plsc.parallel_loop(0, LUT_SIZE // L, unroll=4)
    def _init(i):
        lut_v[pl.ds(i * L, L)] = default

    @plsc.parallel_loop(0, LUT_SIZE // L, unroll=4)
    def _scat(j):
        k = keys_v[pl.ds(j * L, L)]
        v = vals_v[pl.ds(j * L, L)]
        valid = (j * L + lane < VOCAB) & (k >= 0) & (k < LUT_SIZE)
        plsc.store_scatter(lut_v, [k], v, mask=valid)

    head = lut_v[pl.ds(0, L)]
    lut_v[pl.ds(0, L)] = jnp.where(lane == MASK_VALUE, 0, head)

    for s in range(NSLOT):
        if s + 2 < NSLOT:
            if s + 2 >= NBUF:
                out_copy(s + 2 - NBUF, make_only=True).wait()
            in_copy(s + 2)

        in_copy(s, make_only=True).wait()
        buf = bufs[s % NBUF]
        n = TC if s < FULL_RPW else QUART

        @plsc.parallel_loop(0, n // L, unroll=8)
        def _gather(i, buf=buf):
            x = buf[pl.ds(i * L, L)]
            buf[pl.ds(i * L, L)] = plsc.load_gather(lut_v, [x])

        out_copy(s)

    for s in range(max(0, NSLOT - NBUF), NSLOT):
        out_copy(s, make_only=True).wait()


_lookup = functools.partial(
    pl.kernel,
    out_type=jax.ShapeDtypeStruct((TR, TC), jnp.int32),
    mesh=plsc.VectorSubcoreMesh(core_axis_name="c", subcore_axis_name="s"),
    scratch_types=[
        pltpu.VMEM((LUT_SIZE,), jnp.int32),
        pltpu.VMEM((LUT_SIZE,), jnp.int32),
        pltpu.VMEM((LUT_SIZE,), jnp.int32),
        [pltpu.VMEM((TC,), jnp.int32) for _ in range(NBUF)],
        [pltpu.SemaphoreType.DMA for _ in range(3)],
        pltpu.SemaphoreType.DMA,
    ],
    compiler_params=pltpu.CompilerParams(needs_layout_passes=False,
                                         skip_device_barrier=True,
                                         disable_bounds_checks=True,
                                         disable_semaphore_checks=True),
)(_body)


@jax.jit
def kernel(inputs, keys, vals):
    out = _lookup(inputs.T, keys, vals)
    return out.T
